# baseline (speedup 1.0000x reference)
import functools

import jax
import jax.numpy as jnp
from jax import lax
from jax.experimental import pallas as pl
from jax.experimental.pallas import tpu as pltpu
from jax.experimental.pallas import tpu_sc as plsc

ROWS = 128
N = 32768
K = 16384
L = 16

_NC = 2
_NS = 16
NW = _NC * _NS
KPW = K // NW
_IB = 128
_NIB = KPW // _IB
SLICE = N // _NS


def _make_count_kernel():
    mesh = plsc.VectorSubcoreMesh(core_axis_name="c", subcore_axis_name="s")

    @functools.partial(
        pl.kernel,
        mesh=mesh,
        out_type=jax.ShapeDtypeStruct((_NC, N), jnp.float32),
        scratch_types=[
            pltpu.VMEM((_NIB, _IB), jnp.int32),
            pltpu.VMEM((_IB,), jnp.float32),
            pltpu.VMEM((SLICE,), jnp.float32),
            pltpu.VMEM_SHARED((N,), jnp.float32),
            pltpu.SemaphoreType.DMA,
        ],
        compiler_params=pltpu.CompilerParams(needs_layout_passes=False),
    )
    def count_kernel(idx_hbm, cnt_hbm, idx_v, ones_v, zeros_v, cnt_sh, sem):
        cid = lax.axis_index("c")
        sid = lax.axis_index("s")
        wid = sid * _NC + cid

        idx_cp = pltpu.make_async_copy(idx_hbm.at[wid], idx_v, sem)
        idx_cp.start()

        ones16 = jnp.ones((L,), jnp.float32)
        zeros16 = jnp.zeros((L,), jnp.float32)

        def fill_ones(i, _):
            ones_v[pl.ds(i * L, L)] = ones16
            return _

        lax.fori_loop(0, _IB // L, fill_ones, None, unroll=8)

        def fill_zeros(i, _):
            zeros_v[pl.ds(i * L, L)] = zeros16
            return _

        lax.fori_loop(0, SLICE // L, fill_zeros, None, unroll=8)

        pltpu.sync_copy(zeros_v, cnt_sh.at[pl.ds(sid * SLICE, SLICE)])
        idx_cp.wait()
        plsc.subcore_barrier()

        cps = [
            pltpu.async_copy(ones_v, cnt_sh.at[idx_v.at[j]], sem, add=True)
            for j in range(_NIB)
        ]
        for cp in cps:
            cp.wait()
        plsc.subcore_barrier()

        pltpu.sync_copy(
            cnt_sh.at[pl.ds(sid * SLICE, SLICE)],
            cnt_hbm.at[cid, pl.ds(sid * SLICE, SLICE)],
        )

    return count_kernel


_count_kernel = _make_count_kernel()

_RBLK = 64


def _mul_body(x_ref, c_ref, o_ref, m_ref):
    @pl.when(pl.program_id(0) == 0)
    def _():
        m_ref[...] = jnp.minimum(c_ref[0, :] + c_ref[1, :], 1.0)[None, :]

    o_ref[...] = x_ref[...] * m_ref[...]


def kernel(x, retain_idx):
    counts = _count_kernel(retain_idx.reshape(NW, _NIB, _IB))
    out = pl.pallas_call(
        _mul_body,
        grid=(ROWS // _RBLK,),
        in_specs=[
            pl.BlockSpec((_RBLK, N), lambda j: (j, 0)),
            pl.BlockSpec((_NC, N), lambda j: (0, 0)),
        ],
        out_specs=pl.BlockSpec((_RBLK, N), lambda j: (j, 0)),
        out_shape=jax.ShapeDtypeStruct((ROWS, N), jnp.float32),
        scratch_shapes=[pltpu.VMEM((1, N), jnp.float32)],
    )(x, counts)
    return out


# speedup vs baseline: 1.0345x; 1.0345x over previous
import functools

import jax
import jax.numpy as jnp
from jax import lax
from jax.experimental import pallas as pl
from jax.experimental.pallas import tpu as pltpu
from jax.experimental.pallas import tpu_sc as plsc

ROWS = 128
N = 32768
K = 16384
L = 16

_NC = 1
_NS = 16
NW = _NC * _NS
KPW = K // NW
_IB = 128
_NIB = KPW // _IB
SLICE = N // _NS


def _make_count_kernel():
    mesh = plsc.VectorSubcoreMesh(core_axis_name="c", subcore_axis_name="s", num_cores=1)

    @functools.partial(
        pl.kernel,
        mesh=mesh,
        out_type=jax.ShapeDtypeStruct((_NC, N), jnp.float32),
        scratch_types=[
            pltpu.VMEM((_NIB, _IB), jnp.int32),
            pltpu.VMEM((_IB,), jnp.float32),
            pltpu.VMEM((SLICE,), jnp.float32),
            pltpu.VMEM_SHARED((N,), jnp.float32),
            pltpu.SemaphoreType.DMA,
        ],
        compiler_params=pltpu.CompilerParams(needs_layout_passes=False),
    )
    def count_kernel(idx_hbm, cnt_hbm, idx_v, ones_v, zeros_v, cnt_sh, sem):
        cid = lax.axis_index("c")
        sid = lax.axis_index("s")
        wid = sid * _NC + cid

        idx_cp = pltpu.make_async_copy(idx_hbm.at[wid], idx_v, sem)
        idx_cp.start()

        ones16 = jnp.ones((L,), jnp.float32)
        zeros16 = jnp.zeros((L,), jnp.float32)

        def fill_ones(i, _):
            ones_v[pl.ds(i * L, L)] = ones16
            return _

        lax.fori_loop(0, _IB // L, fill_ones, None, unroll=8)

        def fill_zeros(i, _):
            zeros_v[pl.ds(i * L, L)] = zeros16
            return _

        lax.fori_loop(0, SLICE // L, fill_zeros, None, unroll=8)

        pltpu.sync_copy(zeros_v, cnt_sh.at[pl.ds(sid * SLICE, SLICE)])
        idx_cp.wait()
        plsc.subcore_barrier()

        cps = [
            pltpu.async_copy(ones_v, cnt_sh.at[idx_v.at[j]], sem, add=True)
            for j in range(_NIB)
        ]
        for cp in cps:
            cp.wait()
        plsc.subcore_barrier()

        pltpu.sync_copy(
            cnt_sh.at[pl.ds(sid * SLICE, SLICE)],
            cnt_hbm.at[cid, pl.ds(sid * SLICE, SLICE)],
        )

    return count_kernel


_count_kernel = _make_count_kernel()

_RBLK = 64


def _mul_body(x_ref, c_ref, o_ref, m_ref):
    @pl.when(pl.program_id(0) == 0)
    def _():
        m_ref[...] = jnp.minimum(c_ref[0, :], 1.0)[None, :]

    o_ref[...] = x_ref[...] * m_ref[...]


def kernel(x, retain_idx):
    counts = _count_kernel(retain_idx.reshape(NW, _NIB, _IB))
    out = pl.pallas_call(
        _mul_body,
        grid=(ROWS // _RBLK,),
        in_specs=[
            pl.BlockSpec((_RBLK, N), lambda j: (j, 0)),
            pl.BlockSpec((_NC, N), lambda j: (0, 0)),
        ],
        out_specs=pl.BlockSpec((_RBLK, N), lambda j: (j, 0)),
        out_shape=jax.ShapeDtypeStruct((ROWS, N), jnp.float32),
        scratch_shapes=[pltpu.VMEM((1, N), jnp.float32)],
    )(x, counts)
    return out
